# baseline (speedup 1.0000x reference)
import jax
import jax.numpy as jnp
from jax import lax
from jax.experimental import pallas as pl

B, S, H = 64, 512, 768
V, VD = 196, 2048
MAXP, TV = 512, 2
EPS = 1e-12


def _ln(x, gamma, beta):
    mean = jnp.mean(x, axis=-1, keepdims=True)
    xc = x - mean
    var = jnp.mean(xc * xc, axis=-1, keepdims=True)
    return xc * lax.rsqrt(var + EPS) * gamma + beta


def _body(tt_ids_ref, vtt_ids_ref, ita0_ref, ita1_ref, pos_ref, ttt_ref,
          vttt_ref, vpt0_ref, w_ref, bias_ref, gamma_ref, beta_ref,
          ie_ref, ve_ref, out_emb_ref, out_pos_ref):
    b = pl.program_id(0)
    gamma = gamma_ref[0]
    beta = beta_ref[0]

    ids = tt_ids_ref[b]
    m1 = (ids == 1).astype(jnp.float32)[:, None]
    tte = ttt_ref[0] * (1.0 - m1) + ttt_ref[1] * m1
    out_emb_ref[0, :S, :] = _ln(ie_ref[0] + tte, gamma, beta)
    out_pos_ref[0, :S, :] = pos_ref[:, :]

    vids = vtt_ids_ref[b]
    vm1 = (vids == 1).astype(jnp.float32)[:, None]
    vtt = vttt_ref[0] * (1.0 - vm1) + vttt_ref[1] * vm1
    vis = jnp.dot(ve_ref[0].astype(jnp.bfloat16), w_ref[:, :].astype(jnp.bfloat16),
                  preferred_element_type=jnp.float32) + bias_ref[0]
    out_emb_ref[0, S:, :] = _ln(vis + vtt, gamma, beta)

    i0 = ita0_ref[b]
    i1 = ita1_ref[b]
    iota = lax.broadcasted_iota(jnp.int32, (V, MAXP), 1)
    cnt = ((i0[:, None] == iota).astype(jnp.float32) +
           (i1[:, None] == iota).astype(jnp.float32))
    vpe = jnp.dot(cnt, pos_ref[:, :],
                  preferred_element_type=jnp.float32) * 0.5 + vpt0_ref[0]
    out_pos_ref[0, S:, :] = vpe


def kernel(inputs_embeds, token_type_ids, visual_embeds, visual_token_type_ids,
           image_text_alignment, pos_table, tok_type_table, vis_tok_type_table,
           vis_pos_table, vis_proj_W, vis_proj_b, ln_gamma, ln_beta):
    ita0 = image_text_alignment[:, :, 0]
    ita1 = image_text_alignment[:, :, 1]
    out_shape = jax.ShapeDtypeStruct((B, S + V, H), jnp.float32)
    out_emb, out_pos = pl.pallas_call(
        _body,
        grid=(B,),
        in_specs=[
            pl.BlockSpec((B, S), lambda b: (0, 0)),
            pl.BlockSpec((B, V), lambda b: (0, 0)),
            pl.BlockSpec((B, V), lambda b: (0, 0)),
            pl.BlockSpec((B, V), lambda b: (0, 0)),
            pl.BlockSpec((MAXP, H), lambda b: (0, 0)),
            pl.BlockSpec((TV, H), lambda b: (0, 0)),
            pl.BlockSpec((TV, H), lambda b: (0, 0)),
            pl.BlockSpec((1, H), lambda b: (0, 0)),
            pl.BlockSpec((VD, H), lambda b: (0, 0)),
            pl.BlockSpec((1, H), lambda b: (0, 0)),
            pl.BlockSpec((1, H), lambda b: (0, 0)),
            pl.BlockSpec((1, H), lambda b: (0, 0)),
            pl.BlockSpec((1, S, H), lambda b: (b, 0, 0)),
            pl.BlockSpec((1, V, VD), lambda b: (b, 0, 0)),
        ],
        out_specs=[
            pl.BlockSpec((1, S + V, H), lambda b: (b, 0, 0)),
            pl.BlockSpec((1, S + V, H), lambda b: (b, 0, 0)),
        ],
        out_shape=[out_shape, out_shape],
    )(
        token_type_ids.astype(jnp.int32), visual_token_type_ids.astype(jnp.int32),
        ita0.astype(jnp.int32), ita1.astype(jnp.int32),
        pos_table, tok_type_table, vis_tok_type_table,
        vis_pos_table[0:1], vis_proj_W,
        vis_proj_b.reshape(1, H), ln_gamma.reshape(1, H), ln_beta.reshape(1, H),
        inputs_embeds, visual_embeds,
    )
    return (out_emb, out_pos)


# speedup vs baseline: 1.0284x; 1.0284x over previous
import jax
import jax.numpy as jnp
from jax import lax
from jax.experimental import pallas as pl

B, S, H = 64, 512, 768
V, VD = 196, 2048
MAXP, TV = 512, 2
EPS = 1e-12
NB = 2


def _ln(x, gamma, beta):
    mean = jnp.mean(x, axis=-1, keepdims=True)
    xc = x - mean
    var = jnp.mean(xc * xc, axis=-1, keepdims=True)
    return xc * lax.rsqrt(var + EPS) * gamma + beta


def _body(tt_ids_ref, vtt_ids_ref, ita0_ref, ita1_ref, pos_ref, ttt_ref,
          vttt_ref, vpt0_ref, w_ref, bias_ref, gamma_ref, beta_ref,
          ie_ref, ve_ref, out_emb_ref, out_pos_ref):
    gamma = gamma_ref[0]
    beta = beta_ref[0]

    ids = tt_ids_ref[0]
    m1 = (ids == 1).astype(jnp.float32)[:, :, None]
    tte = ttt_ref[0] * (1.0 - m1) + ttt_ref[1] * m1
    out_emb_ref[:, :S, :] = _ln(ie_ref[:, :, :] + tte, gamma, beta)
    out_pos_ref[:, :S, :] = jnp.broadcast_to(pos_ref[:, :], (NB, S, H))

    vids = vtt_ids_ref[0]
    w = w_ref[:, :].astype(jnp.bfloat16)
    iota = lax.broadcasted_iota(jnp.int32, (V, MAXP), 1)
    for k in range(NB):
        vm1 = (vids[k] == 1).astype(jnp.float32)[:, None]
        vtt = vttt_ref[0] * (1.0 - vm1) + vttt_ref[1] * vm1
        vis = jnp.dot(ve_ref[k].astype(jnp.bfloat16), w,
                      preferred_element_type=jnp.float32) + bias_ref[0]
        out_emb_ref[k, S:, :] = _ln(vis + vtt, gamma, beta)

        i0 = ita0_ref[0, k]
        i1 = ita1_ref[0, k]
        cnt = ((i0[:, None] == iota).astype(jnp.float32) +
               (i1[:, None] == iota).astype(jnp.float32))
        vpe = jnp.dot(cnt, pos_ref[:, :],
                      preferred_element_type=jnp.float32) * 0.5
        out_pos_ref[k, S:, :] = vpe + vpt0_ref[0]


def kernel(inputs_embeds, token_type_ids, visual_embeds, visual_token_type_ids,
           image_text_alignment, pos_table, tok_type_table, vis_tok_type_table,
           vis_pos_table, vis_proj_W, vis_proj_b, ln_gamma, ln_beta):
    ita0 = image_text_alignment[:, :, 0]
    ita1 = image_text_alignment[:, :, 1]
    out_shape = jax.ShapeDtypeStruct((B, S + V, H), jnp.float32)
    out_emb, out_pos = pl.pallas_call(
        _body,
        grid=(B // NB,),
        in_specs=[
            pl.BlockSpec((1, NB, S), lambda g: (g, 0, 0)),
            pl.BlockSpec((1, NB, V), lambda g: (g, 0, 0)),
            pl.BlockSpec((1, NB, V), lambda g: (g, 0, 0)),
            pl.BlockSpec((1, NB, V), lambda g: (g, 0, 0)),
            pl.BlockSpec((MAXP, H), lambda g: (0, 0)),
            pl.BlockSpec((TV, H), lambda g: (0, 0)),
            pl.BlockSpec((TV, H), lambda g: (0, 0)),
            pl.BlockSpec((1, H), lambda g: (0, 0)),
            pl.BlockSpec((VD, H), lambda g: (0, 0)),
            pl.BlockSpec((1, H), lambda g: (0, 0)),
            pl.BlockSpec((1, H), lambda g: (0, 0)),
            pl.BlockSpec((1, H), lambda g: (0, 0)),
            pl.BlockSpec((NB, S, H), lambda g: (g, 0, 0)),
            pl.BlockSpec((NB, V, VD), lambda g: (g, 0, 0)),
        ],
        out_specs=[
            pl.BlockSpec((NB, S + V, H), lambda g: (g, 0, 0)),
            pl.BlockSpec((NB, S + V, H), lambda g: (g, 0, 0)),
        ],
        out_shape=[out_shape, out_shape],
    )(
        token_type_ids.astype(jnp.int32).reshape(B // NB, NB, S),
        visual_token_type_ids.astype(jnp.int32).reshape(B // NB, NB, V),
        ita0.astype(jnp.int32).reshape(B // NB, NB, V),
        ita1.astype(jnp.int32).reshape(B // NB, NB, V),
        pos_table, tok_type_table, vis_tok_type_table,
        vis_pos_table[0:1], vis_proj_W,
        vis_proj_b.reshape(1, H), ln_gamma.reshape(1, H), ln_beta.reshape(1, H),
        inputs_embeds, visual_embeds,
    )
    return (out_emb, out_pos)
